# baseline (speedup 1.0000x reference)
import jax
import jax.numpy as jnp
from jax import lax
from jax.experimental import pallas as pl
from jax.experimental.pallas import tpu as pltpu
from jax.experimental.pallas import tpu_sc as plsc

B = 16384
D = 64
NC = 2
NS = 16
NW = NC * NS
BPW = B // NW
CHUNK = 128
NCH = BPW // CHUNK
L = 16
PR = 128


def _gather_body(ids_hbm, tab_hbm, rows_hbm, ids_v, b0, b1, sem0, sem1,
                 semw0, semw1):
    cid = lax.axis_index("c")
    sid = lax.axis_index("s")
    wid = sid * NC + cid

    pltpu.sync_copy(ids_hbm.at[wid], ids_v)

    buf = [b0, b1]
    sems = [sem0, sem1]
    wsems = [semw0, semw1]

    def fire(j):
        return pltpu.async_copy(tab_hbm.at[ids_v.at[j]], buf[j % 2], sems[j % 2])

    wrs = [None, None]
    pend = fire(0)
    for j in range(NCH):
        nxt = None
        if j + 1 < NCH:
            if wrs[(j + 1) % 2] is not None:
                wrs[(j + 1) % 2].wait()
                wrs[(j + 1) % 2] = None
            nxt = fire(j + 1)
        pend.wait()
        pend = nxt
        wrs[j % 2] = pltpu.async_copy(
            buf[j % 2], rows_hbm.at[pl.ds(wid * BPW + j * CHUNK, CHUNK)],
            wsems[j % 2])
    for w in wrs:
        if w is not None:
            w.wait()


def _dot_body(drows_hbm, rrows_hbm, out_hbm, d0, d1, r0, r1, out_v, sem0, sem1):
    cid = lax.axis_index("c")
    sid = lax.axis_index("s")
    wid = sid * NC + cid

    dbuf = [d0, d1]
    rbuf = [r0, r1]
    sems = [sem0, sem1]

    def fire(j):
        s = sems[j % 2]
        base = wid * BPW + j * CHUNK
        return [pltpu.async_copy(drows_hbm.at[pl.ds(base, CHUNK)], dbuf[j % 2], s),
                pltpu.async_copy(rrows_hbm.at[pl.ds(base, CHUNK)], rbuf[j % 2], s)]

    lanes = lax.broadcasted_iota(jnp.int32, (L,), 0)
    zero_i = jnp.zeros((L,), jnp.int32)

    pend = fire(0)
    for j in range(NCH):
        nxt = fire(j + 1) if j + 1 < NCH else []
        for c in pend:
            c.wait()
        pend = nxt
        db, rb = dbuf[j % 2], rbuf[j % 2]

        def g_body(g, carry):
            row = g * L + lanes

            def d_body(d8, acc):
                for k in range(8):
                    col = zero_i + (d8 * 8 + k)
                    acc = acc + (plsc.load_gather(db, [row, col])
                                 * plsc.load_gather(rb, [row, col]))
                return acc

            acc = lax.fori_loop(0, D // 8, d_body, jnp.zeros((L,), jnp.float32))
            out_v[pl.ds(j * CHUNK + g * L, L)] = acc
            return carry

        lax.fori_loop(0, CHUNK // L, g_body, 0)

    pltpu.sync_copy(out_v, out_hbm.at[pl.ds(wid * BPW, BPW)])


_MESH = plsc.VectorSubcoreMesh(core_axis_name="c", subcore_axis_name="s")
_PARAMS = pltpu.CompilerParams(needs_layout_passes=False, use_tc_tiling_on_sc=False)


def _gather_call(ids3, tab2):
    f = pl.kernel(
        _gather_body,
        out_type=jax.ShapeDtypeStruct((B, PR), jnp.float32),
        mesh=_MESH,
        compiler_params=_PARAMS,
        scratch_types=[
            pltpu.VMEM((NCH, CHUNK), jnp.int32),
            pltpu.VMEM((CHUNK, PR), jnp.float32),
            pltpu.VMEM((CHUNK, PR), jnp.float32),
            pltpu.SemaphoreType.DMA,
            pltpu.SemaphoreType.DMA,
            pltpu.SemaphoreType.DMA,
            pltpu.SemaphoreType.DMA,
        ],
    )
    return f(ids3, tab2)


def _dot_call(drows, rrows):
    f = pl.kernel(
        _dot_body,
        out_type=jax.ShapeDtypeStruct((B,), jnp.float32),
        mesh=_MESH,
        compiler_params=_PARAMS,
        scratch_types=[
            pltpu.VMEM((CHUNK, PR), jnp.float32),
            pltpu.VMEM((CHUNK, PR), jnp.float32),
            pltpu.VMEM((CHUNK, PR), jnp.float32),
            pltpu.VMEM((CHUNK, PR), jnp.float32),
            pltpu.VMEM((BPW,), jnp.float32),
            pltpu.SemaphoreType.DMA,
            pltpu.SemaphoreType.DMA,
        ],
    )
    return f(drows, rrows)


@jax.jit
def _run(did3, rid3, donor_table, receiver_table):
    dtab2 = jnp.pad(donor_table, ((0, 0), (0, PR - D)))
    rtab2 = jnp.pad(receiver_table, ((0, 0), (0, PR - D)))
    drows = _gather_call(did3, dtab2)
    rrows = _gather_call(rid3, rtab2)
    return _dot_call(drows, rrows)


def kernel(donor_ids, receiver_ids, donor_table, receiver_table):
    did3 = donor_ids.astype(jnp.int32).reshape(NW, NCH, CHUNK)
    rid3 = receiver_ids.astype(jnp.int32).reshape(NW, NCH, CHUNK)
    return _run(did3, rid3, donor_table, receiver_table)
